# baseline (speedup 1.0000x reference)
import functools

import jax
import jax.numpy as jnp
from jax import lax
from jax.experimental import pallas as pl
from jax.experimental.pallas import tpu as pltpu
from jax.experimental.pallas import tpu_sc as plsc

N_NODES = 10000
N_FEAT = 128
N_EDGES = 320000

NUM_CORES = 2
NUM_SUBCORES = 16
NUM_WORKERS = NUM_CORES * NUM_SUBCORES
CHUNK = 128
B_CHUNKS = 40
N_BLOCKS = 2
K_CHUNKS = B_CHUNKS * N_BLOCKS
EDGES_PER_WORKER = K_CHUNKS * CHUNK
EDGES_PAD = NUM_WORKERS * EDGES_PER_WORKER
N_PAD = 10240
ROWS_PER_SUBCORE = N_PAD // NUM_SUBCORES
VREGS_PER_ROW = N_FEAT // 16


def _sc_aggregate(x, ei, w, zeros):
    mesh = plsc.VectorSubcoreMesh(core_axis_name="c", subcore_axis_name="s")

    @functools.partial(
        pl.kernel,
        out_type=jax.ShapeDtypeStruct((NUM_CORES * N_PAD, N_FEAT),
                                      jnp.float32),
        mesh=mesh,
        scratch_types=[
            pltpu.VMEM((B_CHUNKS, CHUNK), jnp.int32),
            pltpu.VMEM((B_CHUNKS, CHUNK), jnp.int32),
            pltpu.VMEM((B_CHUNKS, CHUNK), jnp.float32),
            [pltpu.VMEM((CHUNK, N_FEAT), jnp.float32) for _ in range(2)],
            pltpu.VMEM_SHARED((N_PAD, N_FEAT), jnp.float32),
            [pltpu.SemaphoreType.DMA for _ in range(2)],
        ],
    )
    def body(x_hbm, ei_hbm, w_hbm, z_hbm, out_hbm,
             col_v, row_v, w_v, bufs, acc_sh, gsems):
        c = lax.axis_index("c")
        s = lax.axis_index("s")
        wid = c * NUM_SUBCORES + s
        srow = s * ROWS_PER_SUBCORE

        pltpu.sync_copy(z_hbm, acc_sh.at[pl.ds(srow, ROWS_PER_SUBCORE)])
        plsc.subcore_barrier()

        for blk in range(N_BLOCKS):
            kbase = wid * K_CHUNKS + blk * B_CHUNKS
            pltpu.sync_copy(ei_hbm.at[1, pl.ds(kbase, B_CHUNKS)], col_v)
            pltpu.sync_copy(ei_hbm.at[0, pl.ds(kbase, B_CHUNKS)], row_v)
            pltpu.sync_copy(w_hbm.at[pl.ds(kbase, B_CHUNKS)], w_v)
            pltpu.async_copy(x_hbm.at[col_v.at[0]], bufs[0], gsems[0])

            def outer(kk, carry):
                for b2 in range(2):
                    k = kk * 2 + b2
                    buf, gsem = bufs[b2], gsems[b2]
                    nbuf, ngsem = bufs[1 - b2], gsems[1 - b2]
                    pltpu.make_async_copy(x_hbm.at[col_v.at[k]], buf,
                                          gsem).wait()

                    @pl.when(k + 1 < B_CHUNKS)
                    def _():
                        pltpu.async_copy(x_hbm.at[col_v.at[k + 1]], nbuf,
                                         ngsem)

                    def scale_body(g, carry2):
                        wv = w_v[k, pl.ds(g * 16, 16)]
                        for i in range(16):
                            r = g * 16 + i
                            sp = jnp.broadcast_to(wv[i], (16,))
                            for j in range(VREGS_PER_ROW):
                                sl = pl.ds(j * 16, 16)
                                buf[r, sl] = buf[r, sl] * sp
                        return carry2

                    lax.fori_loop(0, CHUNK // 16, scale_body, 0)
                    pltpu.sync_copy(buf, acc_sh.at[row_v.at[k]], add=True)
                return carry

            lax.fori_loop(0, B_CHUNKS // 2, outer, 0)

        plsc.subcore_barrier()
        pltpu.sync_copy(acc_sh.at[pl.ds(srow, ROWS_PER_SUBCORE)],
                        out_hbm.at[pl.ds(c * N_PAD + srow,
                                         ROWS_PER_SUBCORE)])

    return body(x, ei, w, zeros)


def _tc_project(partial, weight):
    blk = 1024
    grid = N_PAD // blk

    def body(p0_ref, p1_ref, w_ref, o_ref):
        s = p0_ref[...] + p1_ref[...]
        o_ref[...] = jnp.dot(s, w_ref[...],
                             preferred_element_type=jnp.float32)

    return pl.pallas_call(
        body,
        grid=(grid,),
        in_specs=[
            pl.BlockSpec((blk, N_FEAT), lambda i: (i, 0)),
            pl.BlockSpec((blk, N_FEAT), lambda i: (i + grid, 0)),
            pl.BlockSpec((N_FEAT, N_FEAT), lambda i: (0, 0)),
        ],
        out_specs=pl.BlockSpec((blk, N_FEAT), lambda i: (i, 0)),
        out_shape=jax.ShapeDtypeStruct((N_PAD, N_FEAT), jnp.float32),
    )(partial, partial, weight)


def kernel(input, edge_index, edge_weight, weight):
    x = input.astype(jnp.float32)
    pad = EDGES_PAD - N_EDGES
    ei = jnp.pad(edge_index.astype(jnp.int32), ((0, 0), (0, pad))).reshape(
        2, -1, CHUNK)
    w = jnp.pad(edge_weight.astype(jnp.float32), (0, pad)).reshape(-1, CHUNK)
    zeros = jnp.zeros((ROWS_PER_SUBCORE, N_FEAT), jnp.float32)
    partial = _sc_aggregate(x, ei, w, zeros)
    out = _tc_project(partial, weight)
    return out[:N_NODES]


# speedup vs baseline: 3.2638x; 3.2638x over previous
import functools

import jax
import jax.numpy as jnp
from jax import lax
from jax.experimental import pallas as pl
from jax.experimental.pallas import tpu as pltpu
from jax.experimental.pallas import tpu_sc as plsc

N_NODES = 10000
N_FEAT = 128
N_EDGES = 320000

NUM_CORES = 2
NUM_SUBCORES = 16
NUM_WORKERS = NUM_CORES * NUM_SUBCORES
CHUNK = 128
B_CHUNKS = 40
N_BLOCKS = 2
K_CHUNKS = B_CHUNKS * N_BLOCKS
EDGES_PER_WORKER = K_CHUNKS * CHUNK
EDGES_PAD = NUM_WORKERS * EDGES_PER_WORKER
N_PAD = 10240
ROWS_PER_SUBCORE = N_PAD // NUM_SUBCORES
VREGS_PER_ROW = N_FEAT // 16


def _sc_aggregate(x, ei, w, zeros):
    mesh = plsc.VectorSubcoreMesh(core_axis_name="c", subcore_axis_name="s")

    @functools.partial(
        pl.kernel,
        out_type=jax.ShapeDtypeStruct((NUM_CORES * N_PAD, N_FEAT),
                                      jnp.float32),
        mesh=mesh,
        scratch_types=[
            pltpu.VMEM((B_CHUNKS, CHUNK), jnp.int32),
            pltpu.VMEM((B_CHUNKS, CHUNK), jnp.int32),
            pltpu.VMEM((B_CHUNKS, CHUNK), jnp.float32),
            [pltpu.VMEM((CHUNK, N_FEAT), jnp.float32) for _ in range(2)],
            pltpu.VMEM_SHARED((N_PAD, N_FEAT), jnp.float32),
            [pltpu.SemaphoreType.DMA for _ in range(2)],
        ],
    )
    def body(x_hbm, ei_hbm, w_hbm, z_hbm, out_hbm,
             col_v, row_v, w_v, bufs, acc_sh, gsems):
        c = lax.axis_index("c")
        s = lax.axis_index("s")
        wid = c * NUM_SUBCORES + s
        srow = s * ROWS_PER_SUBCORE

        pltpu.sync_copy(z_hbm, acc_sh.at[pl.ds(srow, ROWS_PER_SUBCORE)])
        plsc.subcore_barrier()

        for blk in range(N_BLOCKS):
            kbase = wid * K_CHUNKS + blk * B_CHUNKS
            pltpu.sync_copy(ei_hbm.at[1, pl.ds(kbase, B_CHUNKS)], col_v)
            pltpu.sync_copy(ei_hbm.at[0, pl.ds(kbase, B_CHUNKS)], row_v)
            pltpu.sync_copy(w_hbm.at[pl.ds(kbase, B_CHUNKS)], w_v)
            pltpu.async_copy(x_hbm.at[col_v.at[0]], bufs[0], gsems[0])

            def outer(kk, carry):
                for b2 in range(2):
                    k = kk * 2 + b2
                    buf, gsem = bufs[b2], gsems[b2]
                    nbuf, ngsem = bufs[1 - b2], gsems[1 - b2]
                    pltpu.make_async_copy(x_hbm.at[col_v.at[k]], buf,
                                          gsem).wait()

                    @pl.when(k + 1 < B_CHUNKS)
                    def _():
                        pltpu.async_copy(x_hbm.at[col_v.at[k + 1]], nbuf,
                                         ngsem)

                    def scale_body(g, carry2):
                        wv = w_v[k, pl.ds(g * 16, 16)]
                        for i in range(16):
                            r = g * 16 + i
                            sp = jnp.broadcast_to(wv[i], (16,))
                            for j in range(VREGS_PER_ROW):
                                sl = pl.ds(j * 16, 16)
                                buf[r, sl] = buf[r, sl] * sp
                        return carry2

                    lax.fori_loop(0, CHUNK // 16, scale_body, 0)
                    pltpu.sync_copy(buf, acc_sh.at[row_v.at[k]], add=True)
                return carry

            lax.fori_loop(0, B_CHUNKS // 2, outer, 0)

        plsc.subcore_barrier()
        pltpu.sync_copy(acc_sh.at[pl.ds(srow, ROWS_PER_SUBCORE)],
                        out_hbm.at[pl.ds(c * N_PAD + srow,
                                         ROWS_PER_SUBCORE)])

    return body(x, ei, w, zeros)


def _tc_project(partial, weight):
    blk = 1024
    grid = N_PAD // blk

    def body(p0_ref, p1_ref, w_ref, o_ref):
        s = p0_ref[...] + p1_ref[...]
        o_ref[...] = jnp.dot(s, w_ref[...],
                             preferred_element_type=jnp.float32)

    return pl.pallas_call(
        body,
        grid=(grid,),
        in_specs=[
            pl.BlockSpec((blk, N_FEAT), lambda i: (i, 0)),
            pl.BlockSpec((blk, N_FEAT), lambda i: (i + grid, 0)),
            pl.BlockSpec((N_FEAT, N_FEAT), lambda i: (0, 0)),
        ],
        out_specs=pl.BlockSpec((blk, N_FEAT), lambda i: (i, 0)),
        out_shape=jax.ShapeDtypeStruct((N_PAD, N_FEAT), jnp.float32),
    )(partial, partial, weight)


def kernel(input, edge_index, edge_weight, weight):
    x = input.astype(jnp.float32)
    pad = EDGES_PAD - N_EDGES
    ei = jnp.pad(edge_index.astype(jnp.int32), ((0, 0), (0, pad)),
                 mode="wrap").reshape(2, -1, CHUNK)
    w = jnp.pad(edge_weight.astype(jnp.float32), (0, pad)).reshape(-1, CHUNK)
    zeros = jnp.zeros((ROWS_PER_SUBCORE, N_FEAT), jnp.float32)
    partial = _sc_aggregate(x, ei, w, zeros)
    out = _tc_project(partial, weight)
    return out[:N_NODES]
